# baseline (speedup 1.0000x reference)
import numpy as np
import jax
import jax.numpy as jnp
from jax import lax
from functools import partial
from jax.experimental import pallas as pl
from jax.experimental.pallas import tpu as pltpu
from jax.experimental.pallas import tpu_sc as plsc

_TEMPERATURE = 0.12
_MAX_STEP_SIZE = 0.18
_MOMENTUM = 0.45
_EXPLORATION_RATE = 0.45

_R1 = (13, 15, 26, 6)
_R2 = (17, 29, 16, 24)
_TINY = np.float32(np.finfo(np.float32).tiny)


def _threefry2x32(k0, k1, x0, x1):
    signed = np.dtype(x1.dtype).kind == "i"

    def cst(v):
        v &= 0xFFFFFFFF
        return np.uint32(v).astype(np.int32) if signed else np.uint32(v)

    if isinstance(x1, (np.ndarray, np.generic)):
        shrl = lambda x, r: x >> np.uint32(r)
    else:
        shrl = lambda x, r: lax.shift_right_logical(x, cst(r))

    C0 = int(k0) & 0xFFFFFFFF
    C1 = int(k1) & 0xFFFFFFFF
    C2 = C0 ^ C1 ^ 0x1BD11BDA
    ks0, ks1, ks2 = cst(C0), cst(C1), cst(C2)

    def rnds(x0, x1, rots):
        for r in rots:
            x0 = x0 + x1
            x1 = (x1 << cst(r)) | shrl(x1, 32 - r)
            x1 = x1 ^ x0
        return x0, x1

    x0 = x0 + ks0
    x1 = x1 + ks1
    x0, x1 = rnds(x0, x1, _R1)
    x0 = x0 + ks1
    x1 = x1 + cst(C2 + 1)
    x0, x1 = rnds(x0, x1, _R2)
    x0 = x0 + ks2
    x1 = x1 + cst(C0 + 2)
    x0, x1 = rnds(x0, x1, _R1)
    x0 = x0 + ks0
    x1 = x1 + cst(C1 + 3)
    x0, x1 = rnds(x0, x1, _R2)
    x0 = x0 + ks1
    x1 = x1 + cst(C2 + 4)
    x0, x1 = rnds(x0, x1, _R1)
    x0 = x0 + ks2
    x1 = x1 + cst(C0 + 5)
    return x0, x1


def _np_threefry2x32(k0, k1, x0, x1):
    old = np.seterr(over="ignore")
    try:
        out = _threefry2x32(
            np.uint32(k0), np.uint32(k1), np.uint32(x0), np.uint32(x1)
        )
    finally:
        np.seterr(**old)
    return out


_KS_DATA = np.asarray(_np_threefry2x32(0, 42, 0, 1), dtype=np.uint32)


def _sample_kernel(sal_ref, out_ref, idx_ref, *, k0, k1, H, W, CH):
    b = pl.program_id(0)
    n_chunks = H // CH
    base = jnp.uint32(H * W) * jnp.uint32(b)
    jrow = lax.broadcasted_iota(jnp.uint32, (CH, W), 0) * jnp.uint32(W)
    jcol = lax.broadcasted_iota(jnp.uint32, (CH, W), 1)
    jloc = jrow + jcol

    def body(i, carry):
        vmax, vidx = carry
        sal = sal_ref[0, pl.ds(i * CH, CH), :]
        j = jnp.uint32(CH * W) * i.astype(jnp.uint32) + jloc
        p = base + j
        x0, x1 = _threefry2x32(k0, k1, jnp.zeros_like(p), p)
        bits = x0 ^ x1
        fb = (bits >> np.uint32(9)) | np.uint32(0x3F800000)
        f = lax.bitcast_convert_type(fb, jnp.float32) - np.float32(1.0)
        u = jnp.maximum(_TINY, f * (np.float32(1.0) - _TINY) + _TINY)
        g = -jnp.log(-jnp.log(u))
        score = sal / np.float32(_TEMPERATURE) + g
        take = score > vmax
        vmax = jnp.where(take, score, vmax)
        vidx = jnp.where(take, j.astype(jnp.int32), vidx)
        return vmax, vidx

    vmax0 = jnp.full((CH, W), -jnp.inf, dtype=jnp.float32)
    vidx0 = jnp.zeros((CH, W), dtype=jnp.int32)
    vmax, vidx = lax.fori_loop(0, n_chunks, body, (vmax0, vidx0), unroll=4)
    out_ref[0] = vmax
    idx_ref[0] = vidx


_LN2 = np.float32(0.6931471805599453)
_SQRT2 = np.float32(1.4142135623730951)


def _neg_log(u):
    bits = lax.bitcast_convert_type(u, jnp.int32)
    e = (bits >> np.int32(23)) - np.int32(127)
    mbits = (bits & np.int32(0x7FFFFF)) | np.int32(0x3F800000)
    m = lax.bitcast_convert_type(mbits, jnp.float32)
    big = m > _SQRT2
    m = jnp.where(big, m * np.float32(0.5), m)
    k = (e + jnp.where(big, np.int32(1), np.int32(0))).astype(jnp.float32)
    z = (m - np.float32(1.0)) / (m + np.float32(1.0))
    z2 = z * z
    q = z2 * (
        np.float32(1 / 3)
        + z2
        * (
            np.float32(1 / 5)
            + z2 * (np.float32(1 / 7) + z2 * np.float32(1 / 9))
        )
    )
    t2z = z + z
    lnm = t2z + t2z * q
    return (-_LN2) * k - lnm


def _sc_sample_kernel(sal_hbm, omin_hbm, oidx_hbm, buf_ref, ovecf_ref, oveci_ref, *, k0, k1, HW, row0, rpw, chunk, U):
    wid = lax.axis_index("s") * 2 + lax.axis_index("c")
    lane = lax.broadcasted_iota(jnp.int32, (16,), 0)
    n_chunks = HW // chunk
    n_iters = chunk // (16 * U)
    inv_t = np.float32(1.0 / _TEMPERATURE)

    for rr in range(rpw):
        row = wid * rpw + rr
        p_row = np.int32(HW) * (np.int32(row0) + row)

        def chunk_body(c, carry):
            vmin, vidx = carry
            pltpu.sync_copy(sal_hbm.at[row, pl.ds(c * chunk, chunk)], buf_ref)
            cbase = c * chunk

            def it_body(i, carry2):
                vmin2, vidx2 = carry2
                for s in range(U):
                    off = i * (16 * U) + s * 16
                    sal = buf_ref[pl.ds(off, 16)]
                    j = cbase + off + lane
                    p = p_row + j
                    x0, x1 = _threefry2x32(k0, k1, jnp.zeros_like(p), p)
                    bits = x0 ^ x1
                    fb = lax.shift_right_logical(bits, np.int32(9)) | np.int32(
                        0x3F800000
                    )
                    f = lax.bitcast_convert_type(fb, jnp.float32) - np.float32(1.0)
                    u = jnp.maximum(_TINY, f * (np.float32(1.0) - _TINY) + _TINY)
                    en = _neg_log(u)
                    w = jnp.exp(sal * (-inv_t))
                    s_sc = en * w
                    take = s_sc < vmin2
                    vmin2 = jnp.where(take, s_sc, vmin2)
                    vidx2 = jnp.where(take, j, vidx2)
                return vmin2, vidx2

            return lax.fori_loop(0, n_iters, it_body, (vmin, vidx))

        vmin0 = jnp.full((16,), jnp.inf, dtype=jnp.float32)
        vidx0 = jnp.zeros((16,), dtype=jnp.int32)
        vmin, vidx = lax.fori_loop(0, n_chunks, chunk_body, (vmin0, vidx0))
        ovecf_ref[...] = vmin
        oveci_ref[...] = vidx
        pltpu.sync_copy(ovecf_ref, omin_hbm.at[row])
        pltpu.sync_copy(oveci_ref, oidx_hbm.at[row])


def _sc_sample_rows(sal_sc, row0):
    R, HW = sal_sc.shape
    rpw = R // 32
    mesh = plsc.VectorSubcoreMesh(core_axis_name="c", subcore_axis_name="s")
    chunk = 16384
    kern = partial(
        _sc_sample_kernel,
        k0=int(_KS_DATA[0]),
        k1=int(_KS_DATA[1]),
        HW=HW,
        row0=row0,
        rpw=rpw,
        chunk=chunk,
        U=4,
    )
    f = pl.kernel(
        kern,
        out_type=[
            jax.ShapeDtypeStruct((R, 16), jnp.float32),
            jax.ShapeDtypeStruct((R, 16), jnp.int32),
        ],
        mesh=mesh,
        scratch_types=[
            pltpu.VMEM((chunk,), jnp.float32),
            pltpu.VMEM((16,), jnp.float32),
            pltpu.VMEM((16,), jnp.int32),
        ],
    )
    return f(sal_sc)


def _finalize_kernel(vmax_ref, vidx_ref, scmin_ref, scidx_ref, pos_ref, *, H, W):
    v = vmax_ref[...]
    vidx = vidx_ref[...]
    m = jnp.max(v, axis=1, keepdims=True)
    cand = jnp.where(v == m, vidx, jnp.int32(0x7FFFFFFF))
    idx_tc = jnp.min(cand, axis=1)
    sv = scmin_ref[...]
    sidx = scidx_ref[...]
    sm = jnp.min(sv, axis=1, keepdims=True)
    scand = jnp.where(sv == sm, sidx, jnp.int32(0x7FFFFFFF))
    idx_sc = jnp.min(scand, axis=1)
    idx = jnp.concatenate([idx_tc, idx_sc], axis=0)
    y = (idx // W).astype(jnp.float32) / np.float32(max(H - 1, 1))
    x = (idx % W).astype(jnp.float32) / np.float32(max(W - 1, 1))
    pos_ref[...] = jnp.concatenate([x[:, None], y[:, None]], axis=1)


_SC_ROWS = 32


def _sample_positions(sal3):
    B, H, W = sal3.shape
    CH = 16
    B_tc = B - _SC_ROWS
    sc_min, sc_idx = _sc_sample_rows(
        sal3.reshape(B, H * W)[B_tc:], row0=B_tc
    )
    kern = partial(
        _sample_kernel,
        k0=int(_KS_DATA[0]),
        k1=int(_KS_DATA[1]),
        H=H,
        W=W,
        CH=CH,
    )
    vmax, vidx = pl.pallas_call(
        kern,
        grid=(B_tc,),
        in_specs=[pl.BlockSpec((1, H, W), lambda b: (b, 0, 0))],
        out_specs=[
            pl.BlockSpec((1, CH, W), lambda b: (b, 0, 0)),
            pl.BlockSpec((1, CH, W), lambda b: (b, 0, 0)),
        ],
        out_shape=[
            jax.ShapeDtypeStruct((B_tc, CH, W), jnp.float32),
            jax.ShapeDtypeStruct((B_tc, CH, W), jnp.int32),
        ],
    )(sal3[:B_tc])
    vmax = vmax.reshape(B_tc, CH * W)
    vidx = vidx.reshape(B_tc, CH * W)
    pos = pl.pallas_call(
        partial(_finalize_kernel, H=H, W=W),
        out_shape=jax.ShapeDtypeStruct((B, 2), jnp.float32),
    )(vmax, vidx, sc_min, sc_idx)
    return pos


def kernel(saliency_map, prev_pos, prev_direction, step, seq_len):
    B, _, H, W = saliency_map.shape
    rk = jax.random.key(42)
    kc1, ks, kr, kc2 = jax.random.split(rk, 4)
    sal_pos = _sample_positions(saliency_map.reshape(B, H, W))
    rand_pos = jax.random.uniform(kr, (B, 2), dtype=jnp.float32)
    explore = jax.random.uniform(kc1, ()) < _EXPLORATION_RATE
    base_pos = jnp.where(explore, rand_pos, sal_pos)
    momentum_pos = jnp.clip(prev_pos + prev_direction * _MAX_STEP_SIZE, 0.0, 1.0)
    use_mom = jax.random.uniform(kc2, ()) > _EXPLORATION_RATE
    mixed = (1.0 - _MOMENTUM) * base_pos + _MOMENTUM * momentum_pos
    base_pos = jnp.where(use_mom, mixed, base_pos)
    return base_pos


# speedup vs baseline: 1.0276x; 1.0276x over previous
import numpy as np
import jax
import jax.numpy as jnp
from jax import lax
from functools import partial
from jax.experimental import pallas as pl
from jax.experimental.pallas import tpu as pltpu
from jax.experimental.pallas import tpu_sc as plsc

_TEMPERATURE = 0.12
_MAX_STEP_SIZE = 0.18
_MOMENTUM = 0.45
_EXPLORATION_RATE = 0.45

_R1 = (13, 15, 26, 6)
_R2 = (17, 29, 16, 24)
_TINY = np.float32(np.finfo(np.float32).tiny)


def _threefry2x32(k0, k1, x0, x1):
    signed = np.dtype(x1.dtype).kind == "i"

    def cst(v):
        v &= 0xFFFFFFFF
        return np.uint32(v).astype(np.int32) if signed else np.uint32(v)

    if isinstance(x1, (np.ndarray, np.generic)):
        shrl = lambda x, r: x >> np.uint32(r)
    else:
        shrl = lambda x, r: lax.shift_right_logical(x, cst(r))

    C0 = int(k0) & 0xFFFFFFFF
    C1 = int(k1) & 0xFFFFFFFF
    C2 = C0 ^ C1 ^ 0x1BD11BDA
    ks0, ks1, ks2 = cst(C0), cst(C1), cst(C2)

    def rnds(x0, x1, rots):
        for r in rots:
            x0 = x0 + x1
            x1 = (x1 << cst(r)) | shrl(x1, 32 - r)
            x1 = x1 ^ x0
        return x0, x1

    x0 = x0 + ks0
    x1 = x1 + ks1
    x0, x1 = rnds(x0, x1, _R1)
    x0 = x0 + ks1
    x1 = x1 + cst(C2 + 1)
    x0, x1 = rnds(x0, x1, _R2)
    x0 = x0 + ks2
    x1 = x1 + cst(C0 + 2)
    x0, x1 = rnds(x0, x1, _R1)
    x0 = x0 + ks0
    x1 = x1 + cst(C1 + 3)
    x0, x1 = rnds(x0, x1, _R2)
    x0 = x0 + ks1
    x1 = x1 + cst(C2 + 4)
    x0, x1 = rnds(x0, x1, _R1)
    x0 = x0 + ks2
    x1 = x1 + cst(C0 + 5)
    return x0, x1


def _np_threefry2x32(k0, k1, x0, x1):
    old = np.seterr(over="ignore")
    try:
        out = _threefry2x32(
            np.uint32(k0), np.uint32(k1), np.uint32(x0), np.uint32(x1)
        )
    finally:
        np.seterr(**old)
    return out


_KS_DATA = np.asarray(_np_threefry2x32(0, 42, 0, 1), dtype=np.uint32)


def _sample_kernel(sal_ref, out_ref, idx_ref, *, k0, k1, H, W, CH):
    b = pl.program_id(0)
    n_chunks = H // CH
    base = jnp.uint32(H * W) * jnp.uint32(b)
    jrow = lax.broadcasted_iota(jnp.uint32, (CH, W), 0) * jnp.uint32(W)
    jcol = lax.broadcasted_iota(jnp.uint32, (CH, W), 1)
    jloc = jrow + jcol

    def body(i, carry):
        vmax, vidx = carry
        sal = sal_ref[0, pl.ds(i * CH, CH), :]
        j = jnp.uint32(CH * W) * i.astype(jnp.uint32) + jloc
        p = base + j
        x0, x1 = _threefry2x32(k0, k1, jnp.zeros_like(p), p)
        bits = x0 ^ x1
        fb = (bits >> np.uint32(9)) | np.uint32(0x3F800000)
        f = lax.bitcast_convert_type(fb, jnp.float32) - np.float32(1.0)
        u = jnp.maximum(_TINY, f * (np.float32(1.0) - _TINY) + _TINY)
        g = -jnp.log(-jnp.log(u))
        score = sal / np.float32(_TEMPERATURE) + g
        take = score > vmax
        vmax = jnp.where(take, score, vmax)
        vidx = jnp.where(take, j.astype(jnp.int32), vidx)
        return vmax, vidx

    vmax0 = jnp.full((CH, W), -jnp.inf, dtype=jnp.float32)
    vidx0 = jnp.zeros((CH, W), dtype=jnp.int32)
    vmax, vidx = lax.fori_loop(0, n_chunks, body, (vmax0, vidx0), unroll=4)
    out_ref[0] = vmax
    idx_ref[0] = vidx


_LN2 = np.float32(0.6931471805599453)
_SQRT2 = np.float32(1.4142135623730951)


def _neg_log(u):
    bits = lax.bitcast_convert_type(u, jnp.int32)
    e = (bits >> np.int32(23)) - np.int32(127)
    mbits = (bits & np.int32(0x7FFFFF)) | np.int32(0x3F800000)
    m = lax.bitcast_convert_type(mbits, jnp.float32)
    big = m > _SQRT2
    m = jnp.where(big, m * np.float32(0.5), m)
    k = (e + jnp.where(big, np.int32(1), np.int32(0))).astype(jnp.float32)
    z = (m - np.float32(1.0)) / (m + np.float32(1.0))
    z2 = z * z
    q = z2 * (
        np.float32(1 / 3)
        + z2
        * (
            np.float32(1 / 5)
            + z2 * (np.float32(1 / 7) + z2 * np.float32(1 / 9))
        )
    )
    t2z = z + z
    lnm = t2z + t2z * q
    return (-_LN2) * k - lnm


def _sc_sample_kernel(sal_hbm, omin_hbm, oidx_hbm, buf_ref, ovecf_ref, oveci_ref, *, k0, k1, HW, row0, rpw, chunk, U):
    wid = lax.axis_index("s") * 2 + lax.axis_index("c")
    lane = lax.broadcasted_iota(jnp.int32, (16,), 0)
    n_chunks = HW // chunk
    n_iters = chunk // (16 * U)
    inv_t = np.float32(1.0 / _TEMPERATURE)

    for rr in range(rpw):
        lrow = wid * rpw + rr
        row = np.int32(row0) + lrow
        p_row = np.int32(HW) * row

        def chunk_body(c, carry):
            vmin, vidx = carry
            pltpu.sync_copy(sal_hbm.at[row, pl.ds(c * chunk, chunk)], buf_ref)
            cbase = c * chunk

            def it_body(i, carry2):
                vmin2, vidx2 = carry2
                for s in range(U):
                    off = i * (16 * U) + s * 16
                    sal = buf_ref[pl.ds(off, 16)]
                    j = cbase + off + lane
                    p = p_row + j
                    x0, x1 = _threefry2x32(k0, k1, jnp.zeros_like(p), p)
                    bits = x0 ^ x1
                    fb = lax.shift_right_logical(bits, np.int32(9)) | np.int32(
                        0x3F800000
                    )
                    f = lax.bitcast_convert_type(fb, jnp.float32) - np.float32(1.0)
                    u = jnp.maximum(_TINY, f * (np.float32(1.0) - _TINY) + _TINY)
                    en = _neg_log(u)
                    w = jnp.exp(sal * (-inv_t))
                    s_sc = en * w
                    take = s_sc < vmin2
                    vmin2 = jnp.where(take, s_sc, vmin2)
                    vidx2 = jnp.where(take, j, vidx2)
                return vmin2, vidx2

            return lax.fori_loop(0, n_iters, it_body, (vmin, vidx))

        vmin0 = jnp.full((16,), jnp.inf, dtype=jnp.float32)
        vidx0 = jnp.zeros((16,), dtype=jnp.int32)
        vmin, vidx = lax.fori_loop(0, n_chunks, chunk_body, (vmin0, vidx0))
        ovecf_ref[...] = vmin
        oveci_ref[...] = vidx
        pltpu.sync_copy(ovecf_ref, omin_hbm.at[lrow])
        pltpu.sync_copy(oveci_ref, oidx_hbm.at[lrow])


def _sc_sample_rows(sal2d, row0, R):
    _, HW = sal2d.shape
    rpw = R // 32
    mesh = plsc.VectorSubcoreMesh(core_axis_name="c", subcore_axis_name="s")
    chunk = 16384
    kern = partial(
        _sc_sample_kernel,
        k0=int(_KS_DATA[0]),
        k1=int(_KS_DATA[1]),
        HW=HW,
        row0=row0,
        rpw=rpw,
        chunk=chunk,
        U=8,
    )
    f = pl.kernel(
        kern,
        out_type=[
            jax.ShapeDtypeStruct((R, 16), jnp.float32),
            jax.ShapeDtypeStruct((R, 16), jnp.int32),
        ],
        mesh=mesh,
        scratch_types=[
            pltpu.VMEM((chunk,), jnp.float32),
            pltpu.VMEM((16,), jnp.float32),
            pltpu.VMEM((16,), jnp.int32),
        ],
    )
    return f(sal2d)


def _finalize_kernel(vmax_ref, vidx_ref, scmin_ref, scidx_ref, pos_ref, *, H, W):
    v = vmax_ref[...]
    vidx = vidx_ref[...]
    m = jnp.max(v, axis=1, keepdims=True)
    cand = jnp.where(v == m, vidx, jnp.int32(0x7FFFFFFF))
    idx_tc = jnp.min(cand, axis=1)
    sv = scmin_ref[...]
    sidx = scidx_ref[...]
    sm = jnp.min(sv, axis=1, keepdims=True)
    scand = jnp.where(sv == sm, sidx, jnp.int32(0x7FFFFFFF))
    idx_sc = jnp.min(scand, axis=1)
    idx = jnp.concatenate([idx_tc, idx_sc], axis=0)
    y = (idx // W).astype(jnp.float32) / np.float32(max(H - 1, 1))
    x = (idx % W).astype(jnp.float32) / np.float32(max(W - 1, 1))
    pos_ref[...] = jnp.concatenate([x[:, None], y[:, None]], axis=1)


_SC_ROWS = 32


def _sample_positions(sal3):
    B, H, W = sal3.shape
    CH = 16
    B_tc = B - _SC_ROWS
    sc_min, sc_idx = _sc_sample_rows(
        sal3.reshape(B, H * W), row0=B_tc, R=_SC_ROWS
    )
    kern = partial(
        _sample_kernel,
        k0=int(_KS_DATA[0]),
        k1=int(_KS_DATA[1]),
        H=H,
        W=W,
        CH=CH,
    )
    vmax, vidx = pl.pallas_call(
        kern,
        grid=(B_tc,),
        in_specs=[pl.BlockSpec((1, H, W), lambda b: (b, 0, 0))],
        out_specs=[
            pl.BlockSpec((1, CH, W), lambda b: (b, 0, 0)),
            pl.BlockSpec((1, CH, W), lambda b: (b, 0, 0)),
        ],
        out_shape=[
            jax.ShapeDtypeStruct((B_tc, CH, W), jnp.float32),
            jax.ShapeDtypeStruct((B_tc, CH, W), jnp.int32),
        ],
    )(sal3[:B_tc])
    vmax = vmax.reshape(B_tc, CH * W)
    vidx = vidx.reshape(B_tc, CH * W)
    pos = pl.pallas_call(
        partial(_finalize_kernel, H=H, W=W),
        out_shape=jax.ShapeDtypeStruct((B, 2), jnp.float32),
    )(vmax, vidx, sc_min, sc_idx)
    return pos


def kernel(saliency_map, prev_pos, prev_direction, step, seq_len):
    B, _, H, W = saliency_map.shape
    rk = jax.random.key(42)
    kc1, ks, kr, kc2 = jax.random.split(rk, 4)
    sal_pos = _sample_positions(saliency_map.reshape(B, H, W))
    rand_pos = jax.random.uniform(kr, (B, 2), dtype=jnp.float32)
    explore = jax.random.uniform(kc1, ()) < _EXPLORATION_RATE
    base_pos = jnp.where(explore, rand_pos, sal_pos)
    momentum_pos = jnp.clip(prev_pos + prev_direction * _MAX_STEP_SIZE, 0.0, 1.0)
    use_mom = jax.random.uniform(kc2, ()) > _EXPLORATION_RATE
    mixed = (1.0 - _MOMENTUM) * base_pos + _MOMENTUM * momentum_pos
    base_pos = jnp.where(use_mom, mixed, base_pos)
    return base_pos


# speedup vs baseline: 1.0349x; 1.0071x over previous
import numpy as np
import jax
import jax.numpy as jnp
from jax import lax
from functools import partial
from jax.experimental import pallas as pl
from jax.experimental.pallas import tpu as pltpu
from jax.experimental.pallas import tpu_sc as plsc

_TEMPERATURE = 0.12
_MAX_STEP_SIZE = 0.18
_MOMENTUM = 0.45
_EXPLORATION_RATE = 0.45

_R1 = (13, 15, 26, 6)
_R2 = (17, 29, 16, 24)
_TINY = np.float32(np.finfo(np.float32).tiny)


def _threefry2x32(k0, k1, x0, x1):
    signed = np.dtype(x1.dtype).kind == "i"

    def cst(v):
        v &= 0xFFFFFFFF
        return np.uint32(v).astype(np.int32) if signed else np.uint32(v)

    if isinstance(x1, (np.ndarray, np.generic)):
        shrl = lambda x, r: x >> np.uint32(r)
    else:
        shrl = lambda x, r: lax.shift_right_logical(x, cst(r))

    C0 = int(k0) & 0xFFFFFFFF
    C1 = int(k1) & 0xFFFFFFFF
    C2 = C0 ^ C1 ^ 0x1BD11BDA
    ks0, ks1, ks2 = cst(C0), cst(C1), cst(C2)

    def rnds(x0, x1, rots):
        for r in rots:
            x0 = x0 + x1
            x1 = (x1 << cst(r)) | shrl(x1, 32 - r)
            x1 = x1 ^ x0
        return x0, x1

    x0 = x0 + ks0
    x1 = x1 + ks1
    x0, x1 = rnds(x0, x1, _R1)
    x0 = x0 + ks1
    x1 = x1 + cst(C2 + 1)
    x0, x1 = rnds(x0, x1, _R2)
    x0 = x0 + ks2
    x1 = x1 + cst(C0 + 2)
    x0, x1 = rnds(x0, x1, _R1)
    x0 = x0 + ks0
    x1 = x1 + cst(C1 + 3)
    x0, x1 = rnds(x0, x1, _R2)
    x0 = x0 + ks1
    x1 = x1 + cst(C2 + 4)
    x0, x1 = rnds(x0, x1, _R1)
    x0 = x0 + ks2
    x1 = x1 + cst(C0 + 5)
    return x0, x1


def _np_threefry2x32(k0, k1, x0, x1):
    old = np.seterr(over="ignore")
    try:
        out = _threefry2x32(
            np.uint32(k0), np.uint32(k1), np.uint32(x0), np.uint32(x1)
        )
    finally:
        np.seterr(**old)
    return out


_KS_DATA = np.asarray(_np_threefry2x32(0, 42, 0, 1), dtype=np.uint32)


def _sample_kernel(sal_ref, out_ref, idx_ref, *, k0, k1, H, W, CH):
    b = pl.program_id(0)
    n_chunks = H // CH
    base = jnp.uint32(H * W) * jnp.uint32(b)
    jrow = lax.broadcasted_iota(jnp.uint32, (CH, W), 0) * jnp.uint32(W)
    jcol = lax.broadcasted_iota(jnp.uint32, (CH, W), 1)
    jloc = jrow + jcol

    def body(i, carry):
        vmax, vidx = carry
        sal = sal_ref[0, pl.ds(i * CH, CH), :]
        j = jnp.uint32(CH * W) * i.astype(jnp.uint32) + jloc
        p = base + j
        x0, x1 = _threefry2x32(k0, k1, jnp.zeros_like(p), p)
        bits = x0 ^ x1
        fb = (bits >> np.uint32(9)) | np.uint32(0x3F800000)
        f = lax.bitcast_convert_type(fb, jnp.float32) - np.float32(1.0)
        u = jnp.maximum(_TINY, f * (np.float32(1.0) - _TINY) + _TINY)
        g = -jnp.log(-jnp.log(u))
        score = sal / np.float32(_TEMPERATURE) + g
        take = score > vmax
        vmax = jnp.where(take, score, vmax)
        vidx = jnp.where(take, j.astype(jnp.int32), vidx)
        return vmax, vidx

    vmax0 = jnp.full((CH, W), -jnp.inf, dtype=jnp.float32)
    vidx0 = jnp.zeros((CH, W), dtype=jnp.int32)
    vmax, vidx = lax.fori_loop(0, n_chunks, body, (vmax0, vidx0), unroll=4)
    out_ref[0] = vmax
    idx_ref[0] = vidx


_LN2 = np.float32(0.6931471805599453)
_SQRT2 = np.float32(1.4142135623730951)


def _neg_log(u):
    bits = lax.bitcast_convert_type(u, jnp.int32)
    e = (bits >> np.int32(23)) - np.int32(127)
    mbits = (bits & np.int32(0x7FFFFF)) | np.int32(0x3F800000)
    m = lax.bitcast_convert_type(mbits, jnp.float32)
    big = m > _SQRT2
    m = jnp.where(big, m * np.float32(0.5), m)
    k = (e + jnp.where(big, np.int32(1), np.int32(0))).astype(jnp.float32)
    z = (m - np.float32(1.0)) / (m + np.float32(1.0))
    z2 = z * z
    q = z2 * (
        np.float32(1 / 3)
        + z2
        * (
            np.float32(1 / 5)
            + z2 * (np.float32(1 / 7) + z2 * np.float32(1 / 9))
        )
    )
    t2z = z + z
    lnm = t2z + t2z * q
    return (-_LN2) * k - lnm


def _sc_sample_kernel(sal_hbm, omin_hbm, oidx_hbm, buf_ref, ovecf_ref, oveci_ref, *, k0, k1, HW, row0, rpw, chunk, U):
    wid = lax.axis_index("s") * 2 + lax.axis_index("c")
    lane = lax.broadcasted_iota(jnp.int32, (16,), 0)
    n_chunks = HW // chunk
    n_iters = chunk // (16 * U)
    inv_t = np.float32(1.0 / _TEMPERATURE)

    for rr in range(rpw):
        lrow = wid * rpw + rr
        row = np.int32(row0) + lrow
        p_row = np.int32(HW) * row

        def chunk_body(c, carry):
            vmin, vidx = carry
            pltpu.sync_copy(sal_hbm.at[pl.ds(p_row + c * chunk, chunk)], buf_ref)
            cbase = c * chunk

            def it_body(i, carry2):
                vmin2, vidx2 = carry2
                for s in range(U):
                    off = i * (16 * U) + s * 16
                    sal = buf_ref[pl.ds(off, 16)]
                    j = cbase + off + lane
                    p = p_row + j
                    x0, x1 = _threefry2x32(k0, k1, jnp.zeros_like(p), p)
                    bits = x0 ^ x1
                    fb = lax.shift_right_logical(bits, np.int32(9)) | np.int32(
                        0x3F800000
                    )
                    f = lax.bitcast_convert_type(fb, jnp.float32) - np.float32(1.0)
                    u = jnp.maximum(_TINY, f * (np.float32(1.0) - _TINY) + _TINY)
                    en = _neg_log(u)
                    w = jnp.exp(sal * (-inv_t))
                    s_sc = en * w
                    take = s_sc < vmin2
                    vmin2 = jnp.where(take, s_sc, vmin2)
                    vidx2 = jnp.where(take, j, vidx2)
                return vmin2, vidx2

            return lax.fori_loop(0, n_iters, it_body, (vmin, vidx))

        vmin0 = jnp.full((16,), jnp.inf, dtype=jnp.float32)
        vidx0 = jnp.zeros((16,), dtype=jnp.int32)
        vmin, vidx = lax.fori_loop(0, n_chunks, chunk_body, (vmin0, vidx0))
        ovecf_ref[...] = vmin
        oveci_ref[...] = vidx
        pltpu.sync_copy(ovecf_ref, omin_hbm.at[lrow])
        pltpu.sync_copy(oveci_ref, oidx_hbm.at[lrow])


def _sc_sample_rows(sal1d, HW, row0, R):
    rpw = R // 32
    mesh = plsc.VectorSubcoreMesh(core_axis_name="c", subcore_axis_name="s")
    chunk = 16384
    kern = partial(
        _sc_sample_kernel,
        k0=int(_KS_DATA[0]),
        k1=int(_KS_DATA[1]),
        HW=HW,
        row0=row0,
        rpw=rpw,
        chunk=chunk,
        U=4,
    )
    f = pl.kernel(
        kern,
        out_type=[
            jax.ShapeDtypeStruct((R, 16), jnp.float32),
            jax.ShapeDtypeStruct((R, 16), jnp.int32),
        ],
        mesh=mesh,
        scratch_types=[
            pltpu.VMEM((chunk,), jnp.float32),
            pltpu.VMEM((16,), jnp.float32),
            pltpu.VMEM((16,), jnp.int32),
        ],
    )
    return f(sal1d)


def _finalize_kernel(vmax_ref, vidx_ref, scmin_ref, scidx_ref, pos_ref, *, H, W):
    v = vmax_ref[...]
    vidx = vidx_ref[...]
    m = jnp.max(v, axis=1, keepdims=True)
    cand = jnp.where(v == m, vidx, jnp.int32(0x7FFFFFFF))
    idx_tc = jnp.min(cand, axis=1)
    sv = scmin_ref[...]
    sidx = scidx_ref[...]
    sm = jnp.min(sv, axis=1, keepdims=True)
    scand = jnp.where(sv == sm, sidx, jnp.int32(0x7FFFFFFF))
    idx_sc = jnp.min(scand, axis=1)
    idx = jnp.concatenate([idx_tc, idx_sc], axis=0)
    y = (idx // W).astype(jnp.float32) / np.float32(max(H - 1, 1))
    x = (idx % W).astype(jnp.float32) / np.float32(max(W - 1, 1))
    pos_ref[...] = jnp.concatenate([x[:, None], y[:, None]], axis=1)


_SC_ROWS = 32


def _sample_positions(sal3):
    B, H, W = sal3.shape
    CH = 16
    B_tc = B - _SC_ROWS
    sc_min, sc_idx = _sc_sample_rows(
        sal3.reshape(B * H * W), HW=H * W, row0=B_tc, R=_SC_ROWS
    )
    kern = partial(
        _sample_kernel,
        k0=int(_KS_DATA[0]),
        k1=int(_KS_DATA[1]),
        H=H,
        W=W,
        CH=CH,
    )
    vmax, vidx = pl.pallas_call(
        kern,
        grid=(B_tc,),
        in_specs=[pl.BlockSpec((1, H, W), lambda b: (b, 0, 0))],
        out_specs=[
            pl.BlockSpec((1, CH, W), lambda b: (b, 0, 0)),
            pl.BlockSpec((1, CH, W), lambda b: (b, 0, 0)),
        ],
        out_shape=[
            jax.ShapeDtypeStruct((B_tc, CH, W), jnp.float32),
            jax.ShapeDtypeStruct((B_tc, CH, W), jnp.int32),
        ],
    )(sal3[:B_tc])
    vmax = vmax.reshape(B_tc, CH * W)
    vidx = vidx.reshape(B_tc, CH * W)
    pos = pl.pallas_call(
        partial(_finalize_kernel, H=H, W=W),
        out_shape=jax.ShapeDtypeStruct((B, 2), jnp.float32),
    )(vmax, vidx, sc_min, sc_idx)
    return pos


def kernel(saliency_map, prev_pos, prev_direction, step, seq_len):
    B, _, H, W = saliency_map.shape
    rk = jax.random.key(42)
    kc1, ks, kr, kc2 = jax.random.split(rk, 4)
    sal_pos = _sample_positions(saliency_map.reshape(B, H, W))
    rand_pos = jax.random.uniform(kr, (B, 2), dtype=jnp.float32)
    explore = jax.random.uniform(kc1, ()) < _EXPLORATION_RATE
    base_pos = jnp.where(explore, rand_pos, sal_pos)
    momentum_pos = jnp.clip(prev_pos + prev_direction * _MAX_STEP_SIZE, 0.0, 1.0)
    use_mom = jax.random.uniform(kc2, ()) > _EXPLORATION_RATE
    mixed = (1.0 - _MOMENTUM) * base_pos + _MOMENTUM * momentum_pos
    base_pos = jnp.where(use_mom, mixed, base_pos)
    return base_pos
